# baseline (speedup 1.0000x reference)
import jax
import jax.numpy as jnp
from jax.experimental import pallas as pl
from jax.experimental.pallas import tpu as pltpu

_ROWS = 25088
_COLS = 1024
_CH = 512
_NCHUNKS = _ROWS // _CH
_NBUF = 7


def _copy_body(x_hbm, o_hbm, *scratch):
    bufs = scratch[:_NBUF]
    in_sems = scratch[_NBUF:2 * _NBUF]
    out_sems = scratch[2 * _NBUF:]

    def in_copy(i):
        s = i % _NBUF
        return pltpu.make_async_copy(
            x_hbm.at[pl.ds(i * _CH, _CH), :], bufs[s], in_sems[s]
        )

    def out_copy(i):
        s = i % _NBUF
        return pltpu.make_async_copy(
            bufs[s], o_hbm.at[pl.ds(i * _CH, _CH), :], out_sems[s]
        )

    for i in range(min(_NBUF, _NCHUNKS)):
        in_copy(i).start()
    for i in range(_NCHUNKS):
        in_copy(i).wait()
        out_copy(i).start()
        nxt = i + _NBUF
        if nxt < _NCHUNKS:
            out_copy(i).wait()
            in_copy(nxt).start()
    for i in range(max(0, _NCHUNKS - _NBUF), _NCHUNKS):
        out_copy(i).wait()


def kernel(x):
    flat = x.reshape(_ROWS, _COLS)
    out = pl.pallas_call(
        _copy_body,
        in_specs=[pl.BlockSpec(memory_space=pl.ANY)],
        out_specs=pl.BlockSpec(memory_space=pl.ANY),
        out_shape=jax.ShapeDtypeStruct((_ROWS, _COLS), x.dtype),
        scratch_shapes=(
            [pltpu.VMEM((_CH, _COLS), jnp.float32) for _ in range(_NBUF)]
            + [pltpu.SemaphoreType.DMA(()) for _ in range(2 * _NBUF)]
        ),
    )(flat)
    return out.reshape(x.shape)


# speedup vs baseline: 1.4974x; 1.4974x over previous
import functools

import jax
import jax.numpy as jnp
from jax import lax
from jax.experimental import pallas as pl
from jax.experimental.pallas import tpu as pltpu
from jax.experimental.pallas import tpu_sc as plsc

_NC = 2
_NS = 16
_NW = _NC * _NS

_W0 = 64 // _NW
_NBUF = 4
_NCHUNK = 64
_C1 = 128 // _NCHUNK


def _sc_copy(x_hbm, o_hbm, *scratch):
    bufs = scratch[:_NBUF]
    in_sems = scratch[_NBUF:2 * _NBUF]
    out_sems = scratch[2 * _NBUF:]
    wid = lax.axis_index("s") * _NC + lax.axis_index("c")
    base = wid * _W0

    def in_copy(j):
        b = j % _NBUF
        return pltpu.make_async_copy(
            x_hbm.at[pl.ds(base, _W0), pl.ds(j * _C1, _C1)], bufs[b], in_sems[b]
        )

    def out_copy(j):
        b = j % _NBUF
        return pltpu.make_async_copy(
            bufs[b], o_hbm.at[pl.ds(base, _W0), pl.ds(j * _C1, _C1)], out_sems[b]
        )

    for j in range(min(_NBUF, _NCHUNK)):
        in_copy(j).start()
    for j in range(_NCHUNK):
        in_copy(j).wait()
        out_copy(j).start()
        nxt = j + _NBUF
        if nxt < _NCHUNK:
            out_copy(j).wait()
            in_copy(nxt).start()
    for j in range(max(0, _NCHUNK - _NBUF), _NCHUNK):
        out_copy(j).wait()


def kernel(x):
    run = functools.partial(
        pl.kernel,
        mesh=plsc.VectorSubcoreMesh(core_axis_name="c", subcore_axis_name="s"),
        out_type=jax.ShapeDtypeStruct(x.shape, x.dtype),
        scratch_types=(
            [pltpu.VMEM((_W0, _C1, 56, 56), jnp.float32) for _ in range(_NBUF)]
            + [pltpu.SemaphoreType.DMA for _ in range(2 * _NBUF)]
        ),
    )(_sc_copy)
    return run(x)
